# baseline (speedup 1.0000x reference)
import jax
import jax.numpy as jnp
from jax import lax
from jax.experimental import pallas as pl
from jax.experimental.pallas import tpu as pltpu

_NUM_FED = 50
_LANES = 128


def _prep_kernel(labels_ref, tgt_ref, fedw_ref, gum_ref, w_ref, to_ref):
    Bk, Mk = labels_ref.shape
    Cp = fedw_ref.shape[1]
    labels = labels_ref[...]
    tgt = tgt_ref[...]
    m_iota = lax.broadcasted_iota(jnp.int32, (Bk, Mk, Mk), 2)
    eq3 = tgt[:, :, None] == m_iota
    t_o = jnp.sum(jnp.where(eq3, labels[:, None, :], 0), axis=2)
    to_ref[...] = t_o
    c_iota = lax.broadcasted_iota(jnp.int32, (Bk * Mk, Cp), 1)
    hits = (t_o.reshape(Bk * Mk, 1) == c_iota).astype(jnp.float32)
    uniq = jnp.max(hits, axis=0, keepdims=True)
    n_u = jnp.sum(uniq).astype(jnp.int32)
    prob = fedw_ref[...] * (1.0 - uniq)
    g = -gum_ref[...] - jnp.log(prob)
    g_col = g.reshape(Cp, 1)
    j_lt_c = (lax.broadcasted_iota(jnp.int32, (Cp, Cp), 0)
              < lax.broadcasted_iota(jnp.int32, (Cp, Cp), 1))
    before = (g_col < g) | ((g_col == g) & j_lt_c)
    rank = jnp.sum(before.astype(jnp.int32), axis=0, keepdims=True)
    extra = (rank < (_NUM_FED - n_u)).astype(jnp.float32)
    w_ref[...] = jnp.maximum(uniq, extra)


def _make_dense_kernel(q_valid):
    def _dense_kernel(pa_ref, pb_ref, src_ref, to_ref, w_ref, out_ref):
        b = pl.program_id(0)
        Bk, Mk = src_ref.shape
        nimg = pa_ref.shape[0]
        Qh = pa_ref.shape[1]
        Ck = pa_ref.shape[2]
        n_pad = 2 * Qh - q_valid
        q_row_a = lax.broadcasted_iota(jnp.int32, (1, Qh), 1)
        c_row = lax.broadcasted_iota(jnp.int32, (1, Ck), 1)
        row_col = lax.broadcasted_iota(jnp.int32, (Qh, 1), 0)
        ln2 = jnp.float32(0.6931471805599453)
        acc = jnp.zeros((1, 1), jnp.float32)
        for i in range(nimg):
            xa = pa_ref[i]
            xb = jnp.where(row_col < q_valid - Qh, pb_ref[i], 0.0)
            spa = jnp.maximum(xa, 0.0) + 0.6931471805599453 * jnp.log2(
                1.0 + jnp.exp2(jnp.abs(xa) * -1.4426950408889634))
            spb = jnp.maximum(xb, 0.0) + 0.6931471805599453 * jnp.log2(
                1.0 + jnp.exp2(jnp.abs(xb) * -1.4426950408889634))
            cols = (jnp.sum(spa, axis=0, keepdims=True)
                    + jnp.sum(spb, axis=0, keepdims=True))
            w_row = w_ref[...]
            term = (jnp.sum(cols * w_row, keepdims=True)
                    - ln2 * n_pad * jnp.sum(w_row,
                                            keepdims=True)).reshape(1, 1)
            img = b * nimg + i
            src_col = src_ref[pl.ds(img, 1), :].reshape(Mk, 1)
            t_col = to_ref[pl.ds(img, 1), :].reshape(Mk, 1)
            oh_a = (src_col == q_row_a).astype(jnp.float32)
            oh_b = (src_col - Qh == q_row_a).astype(jnp.float32)
            rows = (lax.dot_general(oh_a, xa, (((1,), (0,)), ((), ())),
                                    preferred_element_type=jnp.float32)
                    + lax.dot_general(oh_b, xb, (((1,), (0,)), ((), ())),
                                      preferred_element_type=jnp.float32))
            corr = jnp.sum(jnp.where(t_col == c_row, rows, 0.0),
                           keepdims=True).reshape(1, 1)
            acc = acc + term - corr
        out_ref[...] = acc.reshape(1, 1, 1)
    return _dense_kernel


def kernel(pred_logits, fed_loss_cls_weights, labels, src_idx, tgt_idx,
           num_boxes):
    B_, Q_, C_ = pred_logits.shape
    M_ = labels.shape[1]
    gum = jax.random.gumbel(jax.random.key(42), (C_ + 1,), jnp.float32)[:C_]
    gum = gum.reshape(1, C_)
    fedw = fed_loss_cls_weights.astype(jnp.float32).reshape(1, C_)
    labels_i = labels.astype(jnp.int32)
    src_i = src_idx.astype(jnp.int32)
    tgt_i = tgt_idx.astype(jnp.int32)

    w, t_o = pl.pallas_call(
        _prep_kernel,
        out_shape=(
            jax.ShapeDtypeStruct((1, C_), jnp.float32),
            jax.ShapeDtypeStruct((B_, M_), jnp.int32),
        ),
    )(labels_i, tgt_i, fedw, gum)

    nimg = 2
    q_half = ((Q_ // 2 + 7) // 8) * 8
    if 2 * q_half < Q_:
        q_half += 8
    partials = pl.pallas_call(
        _make_dense_kernel(Q_),
        grid=(B_ // nimg,),
        in_specs=[
            pl.BlockSpec((nimg, q_half, C_), lambda b: (b, 0, 0)),
            pl.BlockSpec((nimg, q_half, C_), lambda b: (b, 1, 0)),
            pl.BlockSpec((B_, M_), lambda b: (0, 0)),
            pl.BlockSpec((B_, M_), lambda b: (0, 0)),
            pl.BlockSpec((1, C_), lambda b: (0, 0)),
        ],
        out_specs=pl.BlockSpec((1, 1, 1), lambda b: (b, 0, 0)),
        out_shape=jax.ShapeDtypeStruct((B_ // nimg, 1, 1), jnp.float32),
        compiler_params=pltpu.CompilerParams(
            dimension_semantics=("parallel",)),
    )(pred_logits, pred_logits, src_i, t_o, w)
    return jnp.sum(partials) / (B_ * Q_)


# speedup vs baseline: 1.0385x; 1.0385x over previous
import jax
import jax.numpy as jnp
from jax import lax
from jax.experimental import pallas as pl
from jax.experimental.pallas import tpu as pltpu

_NUM_FED = 50


def _make_kernel(q_valid):
    def _fused_kernel(pa_ref, pb_ref, labels_ref, src_ref, tgt_ref, fedw_ref,
                      gum_ref, out_ref, w_ref, to_ref):
        step = pl.program_id(0)
        Bk, Mk = labels_ref.shape
        Ck = fedw_ref.shape[1]
        nimg = pa_ref.shape[0]
        Qh = pa_ref.shape[1]
        n_pad = 2 * Qh - q_valid

        @pl.when(step == 0)
        def _init():
            labels = labels_ref[...]
            tgt = tgt_ref[...]
            m_iota = lax.broadcasted_iota(jnp.int32, (Bk, Mk, Mk), 2)
            eq3 = tgt[:, :, None] == m_iota
            t_o = jnp.sum(jnp.where(eq3, labels[:, None, :], 0), axis=2)
            to_ref[...] = t_o
            c_iota = lax.broadcasted_iota(jnp.int32, (Bk * Mk, Ck), 1)
            hits = (t_o.reshape(Bk * Mk, 1) == c_iota).astype(jnp.float32)
            uniq = jnp.max(hits, axis=0, keepdims=True)
            n_u = jnp.sum(uniq).astype(jnp.int32)
            prob = fedw_ref[...] * (1.0 - uniq)
            g = -gum_ref[...] - jnp.log(prob)
            g_col = g.reshape(Ck, 1)
            j_lt_c = (lax.broadcasted_iota(jnp.int32, (Ck, Ck), 0)
                      < lax.broadcasted_iota(jnp.int32, (Ck, Ck), 1))
            before = (g_col < g) | ((g_col == g) & j_lt_c)
            rank = jnp.sum(before.astype(jnp.int32), axis=0, keepdims=True)
            extra = (rank < (_NUM_FED - n_u)).astype(jnp.float32)
            w_ref[...] = jnp.maximum(uniq, extra)

        q_row = lax.broadcasted_iota(jnp.int32, (1, Qh), 1)
        c_row = lax.broadcasted_iota(jnp.int32, (1, Ck), 1)
        row_col = lax.broadcasted_iota(jnp.int32, (Qh, 1), 0)
        ln2 = jnp.float32(0.6931471805599453)
        w_row = w_ref[...]
        acc = (-ln2 * n_pad * nimg) * jnp.sum(w_row, keepdims=True)
        for i in range(nimg):
            xa = pa_ref[i]
            xb = jnp.where(row_col < q_valid - Qh, pb_ref[i], 0.0)
            spa = jnp.maximum(xa, 0.0) + jnp.log(1.0 + jnp.exp(-jnp.abs(xa)))
            spb = jnp.maximum(xb, 0.0) + jnp.log(1.0 + jnp.exp(-jnp.abs(xb)))
            cols = (jnp.sum(spa, axis=0, keepdims=True)
                    + jnp.sum(spb, axis=0, keepdims=True))
            term = jnp.sum(cols * w_row, keepdims=True)
            img = step * nimg + i
            src_col = src_ref[pl.ds(img, 1), :].reshape(Mk, 1)
            t_col = to_ref[pl.ds(img, 1), :].reshape(Mk, 1)
            oh_a = (src_col == q_row).astype(jnp.float32)
            oh_b = (src_col - Qh == q_row).astype(jnp.float32)
            rows = (lax.dot_general(oh_a, xa, (((1,), (0,)), ((), ())),
                                    preferred_element_type=jnp.float32)
                    + lax.dot_general(oh_b, xb, (((1,), (0,)), ((), ())),
                                      preferred_element_type=jnp.float32))
            corr = jnp.sum(jnp.where(t_col == c_row, rows, 0.0),
                           keepdims=True)
            acc = acc + term - corr
        acc = acc.reshape(1, 1)
        out_ref[...] = jnp.where(step == 0, acc, out_ref[...] + acc)
    return _fused_kernel


def kernel(pred_logits, fed_loss_cls_weights, labels, src_idx, tgt_idx,
           num_boxes):
    B_, Q_, C_ = pred_logits.shape
    M_ = labels.shape[1]
    gum = jax.random.gumbel(jax.random.key(42), (C_ + 1,), jnp.float32)[:C_]
    gum = gum.reshape(1, C_)
    fedw = fed_loss_cls_weights.astype(jnp.float32).reshape(1, C_)
    labels_i = labels.astype(jnp.int32)
    src_i = src_idx.astype(jnp.int32)
    tgt_i = tgt_idx.astype(jnp.int32)

    nimg = 2
    q_half = ((Q_ // 2 + 7) // 8) * 8
    if 2 * q_half < Q_:
        q_half += 8
    out = pl.pallas_call(
        _make_kernel(Q_),
        grid=(B_ // nimg,),
        in_specs=[
            pl.BlockSpec((nimg, q_half, C_), lambda s: (s, 0, 0)),
            pl.BlockSpec((nimg, q_half, C_), lambda s: (s, 1, 0)),
            pl.BlockSpec((B_, M_), lambda s: (0, 0)),
            pl.BlockSpec((B_, M_), lambda s: (0, 0)),
            pl.BlockSpec((B_, M_), lambda s: (0, 0)),
            pl.BlockSpec((1, C_), lambda s: (0, 0)),
            pl.BlockSpec((1, C_), lambda s: (0, 0)),
        ],
        out_specs=pl.BlockSpec((1, 1), lambda s: (0, 0)),
        out_shape=jax.ShapeDtypeStruct((1, 1), jnp.float32),
        scratch_shapes=[
            pltpu.VMEM((1, C_), jnp.float32),
            pltpu.VMEM((B_, M_), jnp.int32),
        ],
    )(pred_logits, pred_logits, labels_i, src_i, tgt_i, fedw, gum)
    return out[0, 0] / (B_ * Q_)


# speedup vs baseline: 1.0390x; 1.0005x over previous
import jax
import jax.numpy as jnp
from jax import lax
from jax.experimental import pallas as pl
from jax.experimental.pallas import tpu as pltpu

_NUM_FED = 50


def _make_kernel(q_valid):
    def _fused_kernel(pa_ref, pb_ref, pc_ref, labels_ref, src_ref, tgt_ref,
                      fedw_ref, gum_ref, out_ref, w_ref, to_ref):
        step = pl.program_id(0)
        Bk, Mk = labels_ref.shape
        Ck = fedw_ref.shape[1]
        nimg = pa_ref.shape[0]
        Qh = pa_ref.shape[1]
        n_pad = 3 * Qh - q_valid

        @pl.when(step == 0)
        def _init():
            labels = labels_ref[...]
            tgt = tgt_ref[...]
            m_iota = lax.broadcasted_iota(jnp.int32, (Bk, Mk, Mk), 2)
            eq3 = tgt[:, :, None] == m_iota
            t_o = jnp.sum(jnp.where(eq3, labels[:, None, :], 0), axis=2)
            to_ref[...] = t_o
            c_iota = lax.broadcasted_iota(jnp.int32, (Bk * Mk, Ck), 1)
            hits = (t_o.reshape(Bk * Mk, 1) == c_iota).astype(jnp.float32)
            uniq = jnp.max(hits, axis=0, keepdims=True)
            n_u = jnp.sum(uniq).astype(jnp.int32)
            prob = fedw_ref[...] * (1.0 - uniq)
            g = -gum_ref[...] - jnp.log(prob)
            g_col = g.reshape(Ck, 1)
            j_lt_c = (lax.broadcasted_iota(jnp.int32, (Ck, Ck), 0)
                      < lax.broadcasted_iota(jnp.int32, (Ck, Ck), 1))
            before = (g_col < g) | ((g_col == g) & j_lt_c)
            rank = jnp.sum(before.astype(jnp.int32), axis=0, keepdims=True)
            extra = (rank < (_NUM_FED - n_u)).astype(jnp.float32)
            w_ref[...] = jnp.maximum(uniq, extra)

        q_row = lax.broadcasted_iota(jnp.int32, (1, Qh), 1)
        c_row = lax.broadcasted_iota(jnp.int32, (1, Ck), 1)
        row_col = lax.broadcasted_iota(jnp.int32, (Qh, 1), 0)
        ln2 = jnp.float32(0.6931471805599453)
        w_row = w_ref[...]
        acc = (-ln2 * n_pad * nimg) * jnp.sum(w_row, keepdims=True)
        for i in range(nimg):
            xa = pa_ref[i]
            xb = pb_ref[i]
            xc = jnp.where(row_col < q_valid - 2 * Qh, pc_ref[i], 0.0)
            spa = jnp.maximum(xa, 0.0) + jnp.log(1.0 + jnp.exp(-jnp.abs(xa)))
            spb = jnp.maximum(xb, 0.0) + jnp.log(1.0 + jnp.exp(-jnp.abs(xb)))
            spc = jnp.maximum(xc, 0.0) + jnp.log(1.0 + jnp.exp(-jnp.abs(xc)))
            cols = (jnp.sum(spa, axis=0, keepdims=True)
                    + jnp.sum(spb, axis=0, keepdims=True)
                    + jnp.sum(spc, axis=0, keepdims=True))
            term = jnp.sum(cols * w_row, keepdims=True)
            img = step * nimg + i
            src_col = src_ref[pl.ds(img, 1), :].reshape(Mk, 1)
            t_col = to_ref[pl.ds(img, 1), :].reshape(Mk, 1)
            oh_a = (src_col == q_row).astype(jnp.float32)
            oh_b = (src_col - Qh == q_row).astype(jnp.float32)
            oh_c = (src_col - 2 * Qh == q_row).astype(jnp.float32)
            rows = (lax.dot_general(oh_a, xa, (((1,), (0,)), ((), ())),
                                    preferred_element_type=jnp.float32)
                    + lax.dot_general(oh_b, xb, (((1,), (0,)), ((), ())),
                                      preferred_element_type=jnp.float32)
                    + lax.dot_general(oh_c, xc, (((1,), (0,)), ((), ())),
                                      preferred_element_type=jnp.float32))
            corr = jnp.sum(jnp.where(t_col == c_row, rows, 0.0),
                           keepdims=True)
            acc = acc + term - corr
        acc = acc.reshape(1, 1)
        out_ref[...] = jnp.where(step == 0, acc, out_ref[...] + acc)
    return _fused_kernel


def kernel(pred_logits, fed_loss_cls_weights, labels, src_idx, tgt_idx,
           num_boxes):
    B_, Q_, C_ = pred_logits.shape
    M_ = labels.shape[1]
    gum = jax.random.gumbel(jax.random.key(42), (C_ + 1,), jnp.float32)[:C_]
    gum = gum.reshape(1, C_)
    fedw = fed_loss_cls_weights.astype(jnp.float32).reshape(1, C_)
    labels_i = labels.astype(jnp.int32)
    src_i = src_idx.astype(jnp.int32)
    tgt_i = tgt_idx.astype(jnp.int32)

    nimg = 2
    q_third = ((Q_ // 3 + 7) // 8) * 8
    while 3 * q_third < Q_:
        q_third += 8
    out = pl.pallas_call(
        _make_kernel(Q_),
        grid=(B_ // nimg,),
        in_specs=[
            pl.BlockSpec((nimg, q_third, C_), lambda s: (s, 0, 0)),
            pl.BlockSpec((nimg, q_third, C_), lambda s: (s, 1, 0)),
            pl.BlockSpec((nimg, q_third, C_), lambda s: (s, 2, 0)),
            pl.BlockSpec((B_, M_), lambda s: (0, 0)),
            pl.BlockSpec((B_, M_), lambda s: (0, 0)),
            pl.BlockSpec((B_, M_), lambda s: (0, 0)),
            pl.BlockSpec((1, C_), lambda s: (0, 0)),
            pl.BlockSpec((1, C_), lambda s: (0, 0)),
        ],
        out_specs=pl.BlockSpec((1, 1), lambda s: (0, 0)),
        out_shape=jax.ShapeDtypeStruct((1, 1), jnp.float32),
        scratch_shapes=[
            pltpu.VMEM((1, C_), jnp.float32),
            pltpu.VMEM((B_, M_), jnp.int32),
        ],
    )(pred_logits, pred_logits, pred_logits, labels_i, src_i, tgt_i, fedw,
      gum)
    return out[0, 0] / (B_ * Q_)
